# baseline (speedup 1.0000x reference)
import functools

import jax
import jax.numpy as jnp
from jax import lax
from jax.experimental import pallas as pl
from jax.experimental.pallas import tpu as pltpu
from jax.experimental.pallas import tpu_sc as plsc

_B = 16384
_D = 64
_NC = 2
_NS = 16
_NW = _NC * _NS
_BPW = _B // _NW
_CHUNK = 128
_NCHUNK = _BPW // _CHUNK
_IDX_ROWS = _B // _CHUNK
_ROWS_PER_W = _IDX_ROWS // _NW


def _sc_body(const_hbm, pred_hbm, hidx_hbm, tidx_hbm, pidx_hbm, out_hbm,
             hidx_v, tidx_v, pidx_v, head_v, tail_v, pred_v, sem):
    wid = lax.axis_index("s") * _NC + lax.axis_index("c")
    base = wid * _BPW
    irow = wid * _ROWS_PER_W

    pltpu.sync_copy(hidx_hbm.at[pl.ds(irow, _ROWS_PER_W)], hidx_v)
    pltpu.sync_copy(tidx_hbm.at[pl.ds(irow, _ROWS_PER_W)], tidx_v)
    pltpu.sync_copy(pidx_hbm.at[pl.ds(irow, _ROWS_PER_W)], pidx_v)

    copies = []
    for j in range(_NCHUNK):
        dst = pl.ds(j * _CHUNK, _CHUNK)
        copies.append(pltpu.async_copy(const_hbm.at[hidx_v.at[j]],
                                       head_v.at[dst], sem))
        copies.append(pltpu.async_copy(const_hbm.at[tidx_v.at[j]],
                                       tail_v.at[dst], sem))
        copies.append(pltpu.async_copy(pred_hbm.at[pidx_v.at[j]],
                                       pred_v.at[dst], sem))
    for cp in copies:
        cp.wait()

    def row(r, carry):
        for c in range(_D // 16):
            s = pl.ds(c * 16, 16)
            head_v[r, s] = head_v[r, s] + pred_v[r, s] - tail_v[r, s]
        return carry

    lax.fori_loop(0, _BPW, row, 0)

    pltpu.sync_copy(head_v, out_hbm.at[pl.ds(base, _BPW)])


@functools.partial(jax.jit, static_argnames=())
def _run(hidx, tidx, pidx, const_table, pred_table):
    mesh = plsc.VectorSubcoreMesh(core_axis_name="c", subcore_axis_name="s")
    kfn = pl.kernel(
        _sc_body,
        out_type=jax.ShapeDtypeStruct((_B, _D), jnp.float32),
        mesh=mesh,
        scratch_types=[
            pltpu.VMEM((_ROWS_PER_W, _CHUNK), jnp.int32),
            pltpu.VMEM((_ROWS_PER_W, _CHUNK), jnp.int32),
            pltpu.VMEM((_ROWS_PER_W, _CHUNK), jnp.int32),
            pltpu.VMEM((_BPW, _D), jnp.float32),
            pltpu.VMEM((_BPW, _D), jnp.float32),
            pltpu.VMEM((_BPW, _D), jnp.float32),
            pltpu.SemaphoreType.DMA,
        ],
        compiler_params=pltpu.CompilerParams(use_tc_tiling_on_sc=False),
    )
    return kfn(const_table, pred_table, hidx, tidx, pidx)


def kernel(predicate_indices, constant_indices, const_table, pred_table):
    hidx = constant_indices[:, 0].reshape(_IDX_ROWS, _CHUNK)
    tidx = constant_indices[:, 1].reshape(_IDX_ROWS, _CHUNK)
    pidx = predicate_indices[:, 0].reshape(_IDX_ROWS, _CHUNK)
    return _run(hidx, tidx, pidx, const_table, pred_table)
